# Initial kernel scaffold; baseline (speedup 1.0000x reference)
import jax
import jax.numpy as jnp
from jax.experimental import pallas as pl


def kernel(x, xchemfea, edge_index, edge_feats, W11, b11, W12, b12, W13, b13, cw1, cb1, W21, b21, W22, b22, W23, b23, cw2, cb2):
    raise NotImplementedError("write your pallas kernel here")


# speedup vs baseline: 3.9403x; 3.9403x over previous
import functools

import jax
import jax.numpy as jnp
from jax import lax
from jax.experimental import pallas as pl
from jax.experimental.pallas import tpu as pltpu
from jax.experimental.pallas import tpu_sc as plsc

NCORE = 2
NSUB = 16
LANES = 16


def _make_sc_agg(N, E, D):
    NW = NCORE * NSUB
    assert E % NW == 0
    EW = E // NW
    C = 80
    assert EW % C == 0
    n_chunks = EW // C
    ZR = 128
    rows_tile = -(-N // NSUB)
    rows_tile = -(-rows_tile // ZR) * ZR
    NP = rows_tile * NSUB
    n_zcopy = rows_tile // ZR

    mesh = plsc.VectorSubcoreMesh(core_axis_name="c", subcore_axis_name="s")

    @functools.partial(
        pl.kernel,
        out_type=jax.ShapeDtypeStruct((NCORE * NP, D), jnp.float32),
        mesh=mesh,
        scratch_types=[
            pltpu.VMEM((C,), jnp.int32),
            pltpu.VMEM((C,), jnp.int32),
            pltpu.VMEM((C,), jnp.float32),
            pltpu.VMEM((C, D), jnp.float32),
            pltpu.VMEM((ZR, D), jnp.float32),
            pltpu.VMEM_SHARED((NP, D), jnp.float32),
            pltpu.SemaphoreType.DMA,
        ],
    )
    def sc_agg(src_hbm, dst_hbm, w_hbm, x_hbm, out_hbm,
               src_v, dst_v, w_v, rows_v, zeros_v, agg_sh, sem):
        cid = lax.axis_index("c")
        sid = lax.axis_index("s")
        wid = cid * NSUB + sid

        @pl.loop(0, ZR)
        def _(r):
            for c8 in range(D // LANES):
                zeros_v[r, pl.ds(c8 * LANES, LANES)] = jnp.zeros(
                    (LANES,), jnp.float32)

        row0 = sid * rows_tile
        for p in range(n_zcopy):
            pltpu.sync_copy(zeros_v, agg_sh.at[pl.ds(row0 + p * ZR, ZR)])
        plsc.subcore_barrier()

        @pl.loop(0, n_chunks)
        def _(j):
            base = pl.multiple_of(wid * EW + j * C, 8)
            pltpu.sync_copy(src_hbm.at[pl.ds(base, C)], src_v)
            pltpu.sync_copy(dst_hbm.at[pl.ds(base, C)], dst_v)
            pltpu.sync_copy(w_hbm.at[pl.ds(base, C)], w_v)
            pltpu.async_copy(x_hbm.at[src_v], rows_v, sem).wait()

            @pl.loop(0, C // LANES)
            def _(g):
                wvec = w_v[pl.ds(pl.multiple_of(g * LANES, 8), LANES)]
                for lane in range(LANES):
                    wspl = jnp.full((LANES,), wvec[lane], jnp.float32)
                    r = g * LANES + lane
                    for c8 in range(D // LANES):
                        sl = pl.ds(c8 * LANES, LANES)
                        rows_v[r, sl] = rows_v[r, sl] * wspl

            pltpu.sync_copy(rows_v, agg_sh.at[dst_v], add=True)

        plsc.subcore_barrier()

        out0 = cid * NP + sid * rows_tile
        for p in range(n_zcopy):
            pltpu.sync_copy(agg_sh.at[pl.ds(row0 + p * ZR, ZR)],
                            out_hbm.at[pl.ds(out0 + p * ZR, ZR)])

    return sc_agg, NP


def _make_tc_dense(N, D, BR):
    assert N % BR == 0
    HI = jax.lax.Precision.HIGHEST

    def body(x_ref, p0_ref, p1_ref, w1_ref, wc_ref, w2_ref, w3_ref,
             b1c_ref, b2_ref, b3_ref, o_ref):
        x = x_ref[...]
        agg = p0_ref[...] + p1_ref[...]
        t1 = jax.lax.dot(x, w1_ref[...], precision=HI)
        tc = jax.lax.dot(agg, wc_ref[...], precision=HI)
        t2 = jax.lax.dot(x, w2_ref[...], precision=HI) + b2_ref[...]
        t3 = jax.lax.dot(x, w3_ref[...], precision=HI) + b3_ref[...]
        o_ref[...] = jnp.maximum(t1 + tc + b1c_ref[...] + t2 * t3, 0.0)

    row_spec = pl.BlockSpec((BR, D), lambda i: (i, 0))
    mat_spec = pl.BlockSpec((D, D), lambda i: (0, 0))
    vec_spec = pl.BlockSpec((1, D), lambda i: (0, 0))
    return pl.pallas_call(
        body,
        grid=(N // BR,),
        in_specs=[row_spec, row_spec, row_spec,
                  mat_spec, mat_spec, mat_spec, mat_spec,
                  vec_spec, vec_spec, vec_spec],
        out_specs=row_spec,
        out_shape=jax.ShapeDtypeStruct((N, D), jnp.float32),
    )


def kernel(x, xchemfea, edge_index, edge_feats,
           W11, b11, W12, b12, W13, b13, cw1, cb1,
           W21, b21, W22, b22, W23, b23, cw2, cb2):
    N, D = x.shape
    E = edge_index.shape[1]
    src = edge_index[0]
    dst = edge_index[1]
    w = edge_feats[:, 0]

    sc_agg, NP = _make_sc_agg(N, E, D)
    tc_dense = _make_tc_dense(N, D, BR=1000)

    def layer(inp, W1, b1, W2, b2, W3, b3, cw, cb):
        parts = sc_agg(src, dst, w, inp)
        b1c = (b1 + cb).reshape(1, D)
        return tc_dense(inp, parts[:N], parts[NP:NP + N],
                        W1, cw.reshape(D, D), W2, W3,
                        b1c, b2.reshape(1, D), b3.reshape(1, D))

    h = layer(x, W11, b11, W12, b12, W13, b13, cw1, cb1)
    return layer(h, W21, b21, W22, b22, W23, b23, cw2, cb2)
